# baseline (speedup 1.0000x reference)
import functools

import jax
import jax.numpy as jnp
from jax import lax
from jax.experimental import pallas as pl
from jax.experimental.pallas import tpu as pltpu
from jax.experimental.pallas import tpu_sc as plsc

N_NODES = 10000
N_EDGES = 160000
D_IN = 256
D_OUT = 256

NC = 2
NS = 16
DH = D_OUT // NC

CHUNK = 128
NCHUNKS = 80
EDGES_PER_TILE = NCHUNKS * CHUNK
N_EDGES_PAD = NS * EDGES_PER_TILE
N_NODES_PAD = 10240
NODES_PER_TILE = N_NODES_PAD // NS
OUT_CHUNK = 128
N_OUT_CHUNKS = NODES_PER_TILE // OUT_CHUNK
FVECS = DH // 16


def _mm_body(x_ref, w_ref, h_ref, ht_ref):
    x = x_ref[...]
    w = w_ref[...]
    h = lax.dot_general(x, w, (((1,), (1,)), ((), ())),
                        preferred_element_type=jnp.float32)
    h_ref[...] = h
    ht_ref[0] = h[:, :DH]
    ht_ref[1] = h[:, DH:]


def _matmul(x, w):
    m_blk = 2000
    grid = (N_NODES // m_blk,)
    return pl.pallas_call(
        _mm_body,
        grid=grid,
        in_specs=[
            pl.BlockSpec((m_blk, D_IN), lambda i: (i, 0)),
            pl.BlockSpec((D_OUT, D_IN), lambda i: (0, 0)),
        ],
        out_specs=[
            pl.BlockSpec((m_blk, D_OUT), lambda i: (i, 0)),
            pl.BlockSpec((NC, m_blk, DH), lambda i: (0, i, 0)),
        ],
        out_shape=[
            jax.ShapeDtypeStruct((N_NODES, D_OUT), jnp.float32),
            jax.ShapeDtypeStruct((NC, N_NODES, DH), jnp.float32),
        ],
    )(x, w)


MBLK = 8
N_MBLK = NCHUNKS // MBLK


def _sc_agg_body(ht_hbm, row_hbm, col_hbm, w_hbm, bias_hbm, a_hbm, act_hbm,
                 agg, col_v, rowb, wb, gbuf, bias_v, a_v, semg, sems, semm):
    c = lax.axis_index("c")
    s = lax.axis_index("s")
    ht_c = ht_hbm.at[c]

    pltpu.sync_copy(bias_hbm.at[c], bias_v)
    pltpu.sync_copy(a_hbm, a_v)
    pltpu.sync_copy(col_hbm.at[s], col_v)
    pltpu.async_copy(row_hbm.at[s].at[pl.ds(0, MBLK)], rowb.at[0], semm)
    pltpu.async_copy(w_hbm.at[s].at[pl.ds(0, MBLK)], wb.at[0], semm)

    def _zrow(r, _):
        for f in range(FVECS):
            gbuf[0, r, pl.ds(f * 16, 16)] = jnp.zeros((16,), jnp.float32)
        return 0
    lax.fori_loop(0, OUT_CHUNK, _zrow, 0)
    def _zcopy(j, _):
        pltpu.sync_copy(gbuf.at[0],
                        agg.at[pl.ds(s * NODES_PER_TILE + j * OUT_CHUNK,
                                     OUT_CHUNK)])
        return 0
    lax.fori_loop(0, N_OUT_CHUNKS, _zcopy, 0)
    pltpu.async_copy(ht_c.at[col_v.at[0]], gbuf.at[0], semg.at[0])
    plsc.subcore_barrier()

    def _mblock(b, _):
        m = lax.rem(b, 2)
        pltpu.make_async_copy(row_hbm.at[s].at[pl.ds(0, MBLK)], rowb.at[m],
                              semm).wait()
        pltpu.make_async_copy(w_hbm.at[s].at[pl.ds(0, MBLK)], wb.at[m],
                              semm).wait()

        def _chunk(k, _):
            g = b * MBLK + k
            buf = lax.rem(g, 2)
            nbuf = 1 - buf
            idx = rowb.at[m].at[k]

            @pl.when(g >= 1)
            def _():
                pltpu.make_async_copy(gbuf.at[nbuf], agg.at[idx],
                                      sems.at[nbuf]).wait()
            @pl.when(g + 1 < NCHUNKS)
            def _():
                pltpu.async_copy(ht_c.at[col_v.at[g + 1]], gbuf.at[nbuf],
                                 semg.at[nbuf])

            pltpu.make_async_copy(ht_c.at[col_v.at[0]], gbuf.at[buf],
                                  semg.at[buf]).wait()
            def _scale(gg, _):
                wvec = wb[m, k, pl.ds(gg * 16, 16)]
                for e in range(16):
                    w = wvec[e]
                    r = gg * 16 + e
                    for f in range(FVECS):
                        sl = pl.ds(f * 16, 16)
                        gbuf[buf, r, sl] = gbuf[buf, r, sl] * w
                return 0
            lax.fori_loop(0, CHUNK // 16, _scale, 0)

            pltpu.async_copy(gbuf.at[buf], agg.at[idx], sems.at[buf],
                             add=True)
            return 0
        lax.fori_loop(0, MBLK, _chunk, 0)

        @pl.when(b + 1 < N_MBLK)
        def _():
            sl_b = pl.ds((b + 1) * MBLK, MBLK)
            pltpu.async_copy(row_hbm.at[s].at[sl_b], rowb.at[1 - m], semm)
            pltpu.async_copy(w_hbm.at[s].at[sl_b], wb.at[1 - m], semm)
        return 0
    lax.fori_loop(0, N_MBLK, _mblock, 0)

    pltpu.make_async_copy(gbuf.at[(NCHUNKS - 1) % 2], agg.at[col_v.at[0]],
                          sems.at[(NCHUNKS - 1) % 2]).wait()
    plsc.subcore_barrier()

    act_c = act_hbm.at[c]
    def _out(j, _):
        base = s * NODES_PER_TILE + j * OUT_CHUNK
        pltpu.sync_copy(agg.at[pl.ds(base, OUT_CHUNK)], gbuf.at[0])

        def _prelu(r, _):
            for f in range(FVECS):
                sl = pl.ds(f * 16, 16)
                v = gbuf[0, r, sl] + bias_v[0, sl]
                a = a_v[...]
                gbuf[0, r, sl] = jnp.where(v >= 0, v, a * v)
            return 0
        lax.fori_loop(0, OUT_CHUNK, _prelu, 0)

        pltpu.sync_copy(gbuf.at[0], act_c.at[pl.ds(base, OUT_CHUNK)])
        return 0
    lax.fori_loop(0, N_OUT_CHUNKS, _out, 0)


_sc_agg = functools.partial(
    pl.kernel,
    out_type=jax.ShapeDtypeStruct((NC, N_NODES_PAD, DH), jnp.float32),
    mesh=plsc.VectorSubcoreMesh(core_axis_name="c", subcore_axis_name="s"),
    scratch_types=[
        pltpu.VMEM_SHARED((N_NODES_PAD, DH), jnp.float32),
        pltpu.VMEM((NCHUNKS, CHUNK), jnp.int32),
        pltpu.VMEM((2, MBLK, CHUNK), jnp.int32),
        pltpu.VMEM((2, MBLK, CHUNK), jnp.float32),
        pltpu.VMEM((2, CHUNK, DH), jnp.float32),
        pltpu.VMEM((8, DH), jnp.float32),
        pltpu.VMEM((16,), jnp.float32),
        pltpu.SemaphoreType.DMA((2,)),
        pltpu.SemaphoreType.DMA((2,)),
        pltpu.SemaphoreType.DMA,
    ],
)(_sc_agg_body)


@jax.jit
def kernel(features, edge_index, edge_weight, W, bias, prelu_a):
    x = features.reshape(N_NODES, D_IN)
    h, ht = _matmul(x, W)

    npad = N_EDGES_PAD - N_EDGES
    pad_idx = (jnp.arange(npad, dtype=jnp.int32) * 37) % N_NODES
    row = jnp.concatenate([edge_index[0].astype(jnp.int32), pad_idx])
    col = jnp.concatenate([edge_index[1].astype(jnp.int32), pad_idx])
    ew = jnp.concatenate([edge_weight.astype(jnp.float32),
                          jnp.zeros((npad,), jnp.float32)])
    row = row.reshape(NS, NCHUNKS, CHUNK)
    col = col.reshape(NS, NCHUNKS, CHUNK)
    ew = ew.reshape(NS, NCHUNKS, CHUNK)
    bias2 = jnp.broadcast_to(bias.reshape(NC, 1, DH), (NC, 8, DH))
    a16 = jnp.broadcast_to(prelu_a.astype(jnp.float32), (16,))

    act2 = _sc_agg(ht, row, col, ew, bias2, a16)
    act = jnp.moveaxis(act2[:, :N_NODES], 0, 1).reshape(1, N_NODES, D_OUT)
    hidden = h.reshape(1, N_NODES, D_OUT)
    return (act, hidden)


# speedup vs baseline: 2.3062x; 2.3062x over previous
import functools

import jax
import jax.numpy as jnp
from jax import lax
from jax.experimental import pallas as pl
from jax.experimental.pallas import tpu as pltpu
from jax.experimental.pallas import tpu_sc as plsc

N_NODES = 10000
N_EDGES = 160000
D_IN = 256
D_OUT = 256

NC = 2
NS = 16
DH = D_OUT // NC

CHUNK = 128
NCHUNKS = 80
EDGES_PER_TILE = NCHUNKS * CHUNK
N_EDGES_PAD = NS * EDGES_PER_TILE
N_NODES_PAD = 10240
NODES_PER_TILE = N_NODES_PAD // NS
OUT_CHUNK = 128
N_OUT_CHUNKS = NODES_PER_TILE // OUT_CHUNK
FVECS = DH // 16


def _mm_body(x_ref, w_ref, h_ref, ht_ref):
    x = x_ref[...]
    w = w_ref[...]
    h = lax.dot_general(x, w, (((1,), (1,)), ((), ())),
                        preferred_element_type=jnp.float32)
    h_ref[...] = h
    ht_ref[0] = h[:, :DH]
    ht_ref[1] = h[:, DH:]


def _matmul(x, w):
    m_blk = 2000
    grid = (N_NODES // m_blk,)
    return pl.pallas_call(
        _mm_body,
        grid=grid,
        in_specs=[
            pl.BlockSpec((m_blk, D_IN), lambda i: (i, 0)),
            pl.BlockSpec((D_OUT, D_IN), lambda i: (0, 0)),
        ],
        out_specs=[
            pl.BlockSpec((m_blk, D_OUT), lambda i: (i, 0)),
            pl.BlockSpec((NC, m_blk, DH), lambda i: (0, i, 0)),
        ],
        out_shape=[
            jax.ShapeDtypeStruct((N_NODES, D_OUT), jnp.float32),
            jax.ShapeDtypeStruct((NC, N_NODES, DH), jnp.float32),
        ],
    )(x, w)


MBLK = 8
N_MBLK = NCHUNKS // MBLK


def _sc_agg_body(ht_hbm, row_hbm, col_hbm, w_hbm, bias_hbm, a_hbm, act_hbm,
                 agg, col_v, rowb, wb, gbuf, bias_v, a_v, semg, sems, semm):
    c = lax.axis_index("c")
    s = lax.axis_index("s")
    ht_c = ht_hbm.at[c]

    pltpu.sync_copy(bias_hbm.at[c], bias_v)
    pltpu.sync_copy(a_hbm, a_v)
    pltpu.sync_copy(col_hbm.at[s], col_v)
    pltpu.async_copy(row_hbm.at[s].at[pl.ds(0, MBLK)], rowb.at[0], semm)
    pltpu.async_copy(w_hbm.at[s].at[pl.ds(0, MBLK)], wb.at[0], semm)

    def _zrow(r, _):
        for f in range(FVECS):
            gbuf[0, r, pl.ds(f * 16, 16)] = jnp.zeros((16,), jnp.float32)
        return 0
    lax.fori_loop(0, OUT_CHUNK, _zrow, 0)
    def _zcopy(j, _):
        pltpu.sync_copy(gbuf.at[0],
                        agg.at[pl.ds(s * NODES_PER_TILE + j * OUT_CHUNK,
                                     OUT_CHUNK)])
        return 0
    lax.fori_loop(0, N_OUT_CHUNKS, _zcopy, 0)
    pltpu.async_copy(ht_c.at[col_v.at[0]], gbuf.at[0], semg.at[0])
    plsc.subcore_barrier()

    def _mblock(b, _):
        m = lax.rem(b, 2)
        pltpu.make_async_copy(row_hbm.at[s].at[pl.ds(0, MBLK)], rowb.at[m],
                              semm).wait()
        pltpu.make_async_copy(w_hbm.at[s].at[pl.ds(0, MBLK)], wb.at[m],
                              semm).wait()

        def _pair(kk, _):
            for half in range(2):
                k = kk * 2 + half
                g = b * MBLK + k
                buf = half
                nbuf = 1 - buf
                idx = rowb.at[m].at[k]

                @pl.when(g >= 1)
                def _():
                    pltpu.make_async_copy(gbuf.at[nbuf], agg.at[idx],
                                          sems.at[nbuf]).wait()
                @pl.when(g + 1 < NCHUNKS)
                def _():
                    pltpu.async_copy(ht_c.at[col_v.at[g + 1]], gbuf.at[nbuf],
                                     semg.at[nbuf])

                pltpu.make_async_copy(ht_c.at[col_v.at[0]], gbuf.at[buf],
                                      semg.at[buf]).wait()
                gb = gbuf.at[buf]
                def _scale(gg, _):
                    wvec = wb[m, k, pl.ds(gg * 16, 16)]
                    for e in range(16):
                        w = wvec[e]
                        r = gg * 16 + e
                        for f in range(FVECS):
                            sl = pl.ds(f * 16, 16)
                            gb[r, sl] = gb[r, sl] * w
                    return 0
                lax.fori_loop(0, CHUNK // 16, _scale, 0)

                pltpu.async_copy(gb, agg.at[idx], sems.at[buf], add=True)
            return 0
        lax.fori_loop(0, MBLK // 2, _pair, 0)

        @pl.when(b + 1 < N_MBLK)
        def _():
            sl_b = pl.ds((b + 1) * MBLK, MBLK)
            pltpu.async_copy(row_hbm.at[s].at[sl_b], rowb.at[1 - m], semm)
            pltpu.async_copy(w_hbm.at[s].at[sl_b], wb.at[1 - m], semm)
        return 0
    lax.fori_loop(0, N_MBLK, _mblock, 0)

    pltpu.make_async_copy(gbuf.at[(NCHUNKS - 1) % 2], agg.at[col_v.at[0]],
                          sems.at[(NCHUNKS - 1) % 2]).wait()
    plsc.subcore_barrier()

    act_c = act_hbm.at[c]
    def _out(j, _):
        base = s * NODES_PER_TILE + j * OUT_CHUNK
        pltpu.sync_copy(agg.at[pl.ds(base, OUT_CHUNK)], gbuf.at[0])

        def _prelu(r, _):
            for f in range(FVECS):
                sl = pl.ds(f * 16, 16)
                v = gbuf[0, r, sl] + bias_v[0, sl]
                a = a_v[...]
                gbuf[0, r, sl] = jnp.where(v >= 0, v, a * v)
            return 0
        lax.fori_loop(0, OUT_CHUNK, _prelu, 0)

        pltpu.sync_copy(gbuf.at[0], act_c.at[pl.ds(base, OUT_CHUNK)])
        return 0
    lax.fori_loop(0, N_OUT_CHUNKS, _out, 0)


_sc_agg = functools.partial(
    pl.kernel,
    out_type=jax.ShapeDtypeStruct((NC, N_NODES_PAD, DH), jnp.float32),
    mesh=plsc.VectorSubcoreMesh(core_axis_name="c", subcore_axis_name="s"),
    scratch_types=[
        pltpu.VMEM_SHARED((N_NODES_PAD, DH), jnp.float32),
        pltpu.VMEM((NCHUNKS, CHUNK), jnp.int32),
        pltpu.VMEM((2, MBLK, CHUNK), jnp.int32),
        pltpu.VMEM((2, MBLK, CHUNK), jnp.float32),
        pltpu.VMEM((2, CHUNK, DH), jnp.float32),
        pltpu.VMEM((8, DH), jnp.float32),
        pltpu.VMEM((16,), jnp.float32),
        pltpu.SemaphoreType.DMA((2,)),
        pltpu.SemaphoreType.DMA((2,)),
        pltpu.SemaphoreType.DMA,
    ],
)(_sc_agg_body)


@jax.jit
def kernel(features, edge_index, edge_weight, W, bias, prelu_a):
    x = features.reshape(N_NODES, D_IN)
    h, ht = _matmul(x, W)

    npad = N_EDGES_PAD - N_EDGES
    pad_idx = (jnp.arange(npad, dtype=jnp.int32) * 37) % N_NODES
    row = jnp.concatenate([edge_index[0].astype(jnp.int32), pad_idx])
    col = jnp.concatenate([edge_index[1].astype(jnp.int32), pad_idx])
    ew = jnp.concatenate([edge_weight.astype(jnp.float32),
                          jnp.zeros((npad,), jnp.float32)])
    row = row.reshape(NS, NCHUNKS, CHUNK)
    col = col.reshape(NS, NCHUNKS, CHUNK)
    ew = ew.reshape(NS, NCHUNKS, CHUNK)
    bias2 = jnp.broadcast_to(bias.reshape(NC, 1, DH), (NC, 8, DH))
    a16 = jnp.broadcast_to(prelu_a.astype(jnp.float32), (16,))

    act2 = _sc_agg(ht, row, col, ew, bias2, a16)
    act = jnp.moveaxis(act2[:, :N_NODES], 0, 1).reshape(1, N_NODES, D_OUT)
    hidden = h.reshape(1, N_NODES, D_OUT)
    return (act, hidden)
